# baseline (speedup 1.0000x reference)
import functools

import jax
import jax.numpy as jnp
from jax import lax
from jax.experimental import pallas as pl
from jax.experimental.pallas import tpu as pltpu
from jax.experimental.pallas import tpu_sc as plsc

NN = 10000
EE = 320000
DD = 128
HH = 64
NCORE = 2
NTILE = 16
NPAD = 10240
RPT = NPAD // NTILE
EPT = EE // NTILE
KCH = 128
NCHUNK = -(-EPT // KCH)
EPAD = NCHUNK * KCH
BN = 400
GRID = NN // BN


NBUF = 3


def _sc_agg_body(y_hbm, src_hbm, dst_hbm, zeros_hbm, out0_hbm, out1_hbm,
                 sidx, didx, rows0, rows1, rows2, agg_sp, sem0, sem1, sem2):
    c = lax.axis_index("c")
    s = lax.axis_index("s")
    rows = (rows0, rows1, rows2)
    sems = (sem0, sem1, sem2)

    pltpu.sync_copy(src_hbm.at[c, s], sidx)
    pltpu.sync_copy(dst_hbm.at[s], didx)
    pltpu.sync_copy(zeros_hbm, agg_sp.at[pl.ds(s * RPT, RPT)])
    plsc.subcore_barrier()

    def start(i, b):
        pltpu.async_copy(y_hbm.at[sidx.at[i]], rows[b], sems[b])

    def wait(b):
        pltpu.make_async_copy(y_hbm.at[pl.ds(0, KCH)], rows[b],
                              sems[b]).wait()

    def process(g, b):
        wait(b)
        pltpu.sync_copy(rows[b], agg_sp.at[didx.at[g]], add=True)

        @pl.when(g + NBUF < NCHUNK)
        def _():
            start(g + NBUF, b)

    for b in range(NBUF):
        start(b, b)

    def chunk_trip(t, carry):
        g = t * NBUF
        for b in range(NBUF):
            process(g + b, b)
        return carry

    lax.fori_loop(0, NCHUNK // NBUF, chunk_trip, 0)
    for r in range(NCHUNK - NCHUNK % NBUF, NCHUNK):
        process(r, r % NBUF)

    plsc.subcore_barrier()

    @pl.when(c == 0)
    def _():
        pltpu.sync_copy(agg_sp.at[pl.ds(s * RPT, RPT)],
                        out0_hbm.at[pl.ds(s * RPT, RPT)])

    @pl.when(c == 1)
    def _():
        pltpu.sync_copy(agg_sp.at[pl.ds(s * RPT, RPT)],
                        out1_hbm.at[pl.ds(s * RPT, RPT)])


_sc_agg = functools.partial(
    pl.kernel,
    mesh=plsc.VectorSubcoreMesh(core_axis_name="c", subcore_axis_name="s"),
    out_type=[jax.ShapeDtypeStruct((NPAD, HH), jnp.float32),
              jax.ShapeDtypeStruct((NPAD, HH), jnp.float32)],
    compiler_params=pltpu.CompilerParams(use_tc_tiling_on_sc=False),
    scratch_types=[
        pltpu.VMEM((NCHUNK, KCH), jnp.int32),
        pltpu.VMEM((NCHUNK, KCH), jnp.int32),
        pltpu.VMEM((KCH, HH), jnp.float32),
        pltpu.VMEM((KCH, HH), jnp.float32),
        pltpu.VMEM((KCH, HH), jnp.float32),
        pltpu.VMEM_SHARED((NPAD, HH), jnp.float32),
        pltpu.SemaphoreType.DMA,
        pltpu.SemaphoreType.DMA,
        pltpu.SemaphoreType.DMA,
    ],
)(_sc_agg_body)


def _tc_first(x_ref, wn_ref, ws_ref, b_ref, y_ref, s_ref):
    x = x_ref[...]
    y_ref[...] = jnp.dot(x, wn_ref[...], preferred_element_type=jnp.float32)
    s_ref[...] = jnp.dot(x, ws_ref[...], preferred_element_type=jnp.float32) + b_ref[...]


def _tc_mid(sp_ref, a0_ref, a1_ref, wn_ref, ws_ref, b_ref, y_ref, s_ref, h_ref):
    agg = jnp.concatenate([a0_ref[...], a1_ref[...]], axis=1)
    h = jnp.maximum(sp_ref[...] + agg, 0.0)
    h_ref[...] = h
    y_ref[...] = jnp.dot(h, wn_ref[...], preferred_element_type=jnp.float32)
    s_ref[...] = jnp.dot(h, ws_ref[...], preferred_element_type=jnp.float32) + b_ref[...]


def _tc_pre(sp_ref, a0_ref, a1_ref, res_ref, wn_ref, ws_ref, b_ref,
            y_ref, s_ref, aux_ref):
    agg = jnp.concatenate([a0_ref[...], a1_ref[...]], axis=1)
    h = jnp.maximum(sp_ref[...] + agg, 0.0)
    aux_ref[...] = h
    x = res_ref[...] + h
    y_ref[...] = jnp.dot(x, wn_ref[...], preferred_element_type=jnp.float32)
    s_ref[...] = jnp.dot(x, ws_ref[...], preferred_element_type=jnp.float32) + b_ref[...]


def _tc_fin(sp_ref, a0_ref, a1_ref, o_ref):
    agg = jnp.concatenate([a0_ref[...], a1_ref[...]], axis=1)
    o_ref[...] = sp_ref[...] + agg


_row = lambda r: (r, 0)
_full = lambda r: (0, 0)
_ND = pl.BlockSpec((BN, DD), _row)
_NH = pl.BlockSpec((BN, HH), _row)
_WB = pl.BlockSpec((DD, DD), _full)
_BB = pl.BlockSpec((1, DD), _full)

_sd_ND = jax.ShapeDtypeStruct((NN, DD), jnp.float32)

_first_call = pl.pallas_call(
    _tc_first, grid=(GRID,),
    in_specs=[_ND, _WB, _WB, _BB],
    out_specs=[_ND, _ND],
    out_shape=[_sd_ND, _sd_ND],
)

_mid_call = pl.pallas_call(
    _tc_mid, grid=(GRID,),
    in_specs=[_ND, _NH, _NH, _WB, _WB, _BB],
    out_specs=[_ND, _ND, _ND],
    out_shape=[_sd_ND, _sd_ND, _sd_ND],
)

_pre_call = pl.pallas_call(
    _tc_pre, grid=(GRID,),
    in_specs=[_ND, _NH, _NH, _ND, _WB, _WB, _BB],
    out_specs=[_ND, _ND, _ND],
    out_shape=[_sd_ND, _sd_ND, _sd_ND],
)

_fin_call = pl.pallas_call(
    _tc_fin, grid=(GRID,),
    in_specs=[_ND, _NH, _NH],
    out_specs=_ND,
    out_shape=_sd_ND,
)


def kernel(features, edges, Wself, Wneigh, b):
    src = edges[0].astype(jnp.int32)
    dst = edges[1].astype(jnp.int32)
    pad = EPAD - EPT
    src_t = jnp.pad(src.reshape(NTILE, EPT), ((0, 0), (0, pad)))
    src_t = src_t.reshape(NTILE, NCHUNK, KCH)
    srcx = jnp.stack([2 * src_t, 2 * src_t + 1])
    dst_t = jnp.pad(dst.reshape(NTILE, EPT), ((0, 0), (0, pad)),
                    constant_values=NN)
    dst_t = dst_t.reshape(NTILE, NCHUNK, KCH)
    zeros = jnp.zeros((RPT, HH), jnp.float32)
    bb = b.reshape(-1, 1, DD)

    def run_sc(y):
        return _sc_agg(y.reshape(NCORE * NN, HH), srcx, dst_t, zeros)

    y, s = _first_call(features, Wneigh[0], Wself[0], bb[0])
    a0, a1 = run_sc(y)
    residual = None
    for i in range(1, 13):
        y, s, h = _mid_call(s, a0, a1, Wneigh[i], Wself[i], bb[i])
        if i == 1:
            residual = h
        a0, a1 = run_sc(y)
    y, s, aux = _pre_call(s, a0, a1, residual, Wneigh[13], Wself[13], bb[13])
    a0, a1 = run_sc(y)
    vertices = _fin_call(s, a0, a1)
    return (vertices, aux)


# speedup vs baseline: 1.1855x; 1.1855x over previous
import functools

import jax
import jax.numpy as jnp
from jax import lax
from jax.experimental import pallas as pl
from jax.experimental.pallas import tpu as pltpu
from jax.experimental.pallas import tpu_sc as plsc

NN = 10000
EE = 320000
DD = 128
HH = 64
NCORE = 2
NTILE = 16
NPAD = 10240
RPT = NPAD // NTILE
EPT = EE // NTILE
KCH = 128
NCHUNK = -(-EPT // KCH)
EPAD = NCHUNK * KCH
BN = 1000
GRID = NN // BN


NBUF = 3


def _sc_agg_body(y_hbm, src_hbm, dst_hbm, zeros_hbm, out0_hbm, out1_hbm,
                 sidx, didx, rows0, rows1, rows2, agg_sp, sem0, sem1, sem2):
    c = lax.axis_index("c")
    s = lax.axis_index("s")
    rows = (rows0, rows1, rows2)
    sems = (sem0, sem1, sem2)

    pltpu.sync_copy(src_hbm.at[s], sidx)
    pltpu.sync_copy(dst_hbm.at[s], didx)
    pltpu.sync_copy(zeros_hbm, agg_sp.at[pl.ds(s * RPT, RPT)])
    plsc.subcore_barrier()

    def start(i, b):
        pltpu.async_copy(y_hbm.at[c].at[sidx.at[i]], rows[b], sems[b])

    def wait(b):
        pltpu.make_async_copy(y_hbm.at[0].at[pl.ds(0, KCH)], rows[b],
                              sems[b]).wait()

    def process(g, b):
        wait(b)
        pltpu.sync_copy(rows[b], agg_sp.at[didx.at[g]], add=True)

        @pl.when(g + NBUF < NCHUNK)
        def _():
            start(g + NBUF, b)

    for b in range(NBUF):
        start(b, b)

    def chunk_trip(t, carry):
        g = t * NBUF
        for b in range(NBUF):
            process(g + b, b)
        return carry

    lax.fori_loop(0, NCHUNK // NBUF, chunk_trip, 0)
    for r in range(NCHUNK - NCHUNK % NBUF, NCHUNK):
        process(r, r % NBUF)

    plsc.subcore_barrier()

    @pl.when(c == 0)
    def _():
        pltpu.sync_copy(agg_sp.at[pl.ds(s * RPT, RPT)],
                        out0_hbm.at[pl.ds(s * RPT, RPT)])

    @pl.when(c == 1)
    def _():
        pltpu.sync_copy(agg_sp.at[pl.ds(s * RPT, RPT)],
                        out1_hbm.at[pl.ds(s * RPT, RPT)])


_sc_agg = functools.partial(
    pl.kernel,
    mesh=plsc.VectorSubcoreMesh(core_axis_name="c", subcore_axis_name="s"),
    out_type=[jax.ShapeDtypeStruct((NPAD, HH), jnp.float32),
              jax.ShapeDtypeStruct((NPAD, HH), jnp.float32)],
    compiler_params=pltpu.CompilerParams(use_tc_tiling_on_sc=False),
    scratch_types=[
        pltpu.VMEM((NCHUNK, KCH), jnp.int32),
        pltpu.VMEM((NCHUNK, KCH), jnp.int32),
        pltpu.VMEM((KCH, HH), jnp.float32),
        pltpu.VMEM((KCH, HH), jnp.float32),
        pltpu.VMEM((KCH, HH), jnp.float32),
        pltpu.VMEM_SHARED((NPAD, HH), jnp.float32),
        pltpu.SemaphoreType.DMA,
        pltpu.SemaphoreType.DMA,
        pltpu.SemaphoreType.DMA,
    ],
)(_sc_agg_body)


def _pack_y(y, y_ref):
    y_ref[0] = y[:, :HH]
    y_ref[1] = y[:, HH:]


def _unpack_agg(a0_ref, a1_ref):
    return jnp.concatenate([a0_ref[...], a1_ref[...]], axis=1)


def _tc_first(x_ref, wn_ref, ws_ref, b_ref, y_ref, s_ref):
    x = x_ref[...]
    _pack_y(jnp.dot(x, wn_ref[...], preferred_element_type=jnp.float32), y_ref)
    s_ref[...] = jnp.dot(x, ws_ref[...], preferred_element_type=jnp.float32) + b_ref[...]


def _tc_mid(sp_ref, a0_ref, a1_ref, wn_ref, ws_ref, b_ref, y_ref, s_ref, h_ref):
    h = jnp.maximum(sp_ref[...] + _unpack_agg(a0_ref, a1_ref), 0.0)
    h_ref[...] = h
    _pack_y(jnp.dot(h, wn_ref[...], preferred_element_type=jnp.float32), y_ref)
    s_ref[...] = jnp.dot(h, ws_ref[...], preferred_element_type=jnp.float32) + b_ref[...]


def _tc_pre(sp_ref, a0_ref, a1_ref, res_ref, wn_ref, ws_ref, b_ref,
            y_ref, s_ref, aux_ref):
    h = jnp.maximum(sp_ref[...] + _unpack_agg(a0_ref, a1_ref), 0.0)
    aux_ref[...] = h
    x = res_ref[...] + h
    _pack_y(jnp.dot(x, wn_ref[...], preferred_element_type=jnp.float32), y_ref)
    s_ref[...] = jnp.dot(x, ws_ref[...], preferred_element_type=jnp.float32) + b_ref[...]


def _tc_fin(sp_ref, a0_ref, a1_ref, o_ref):
    o_ref[...] = sp_ref[...] + _unpack_agg(a0_ref, a1_ref)


_row = lambda r: (r, 0)
_full = lambda r: (0, 0)
_ND = pl.BlockSpec((BN, DD), _row)
_APK = pl.BlockSpec((BN, HH), _row)
_WB = pl.BlockSpec((DD, DD), _full)
_BB = pl.BlockSpec((1, DD), _full)
_Y = pl.BlockSpec((2, BN, HH), lambda r: (0, r, 0))

_sd_ND = jax.ShapeDtypeStruct((NN, DD), jnp.float32)
_sd_Y = jax.ShapeDtypeStruct((2, NN, HH), jnp.float32)

_first_call = pl.pallas_call(
    _tc_first, grid=(GRID,),
    in_specs=[_ND, _WB, _WB, _BB],
    out_specs=[_Y, _ND],
    out_shape=[_sd_Y, _sd_ND],
)

_mid_call = pl.pallas_call(
    _tc_mid, grid=(GRID,),
    in_specs=[_ND, _APK, _APK, _WB, _WB, _BB],
    out_specs=[_Y, _ND, _ND],
    out_shape=[_sd_Y, _sd_ND, _sd_ND],
)

_pre_call = pl.pallas_call(
    _tc_pre, grid=(GRID,),
    in_specs=[_ND, _APK, _APK, _ND, _WB, _WB, _BB],
    out_specs=[_Y, _ND, _ND],
    out_shape=[_sd_Y, _sd_ND, _sd_ND],
)

_fin_call = pl.pallas_call(
    _tc_fin, grid=(GRID,),
    in_specs=[_ND, _APK, _APK],
    out_specs=_ND,
    out_shape=_sd_ND,
)


def kernel(features, edges, Wself, Wneigh, b):
    src = edges[0].astype(jnp.int32)
    dst = edges[1].astype(jnp.int32)
    pad = EPAD - EPT
    src_t = jnp.pad(src.reshape(NTILE, EPT), ((0, 0), (0, pad)))
    src_t = src_t.reshape(NTILE, NCHUNK, KCH)
    dst_t = jnp.pad(dst.reshape(NTILE, EPT), ((0, 0), (0, pad)),
                    constant_values=NN)
    dst_t = dst_t.reshape(NTILE, NCHUNK, KCH)
    zeros = jnp.zeros((RPT, HH), jnp.float32)
    bb = b.reshape(-1, 1, DD)

    def run_sc(y):
        return _sc_agg(y, src_t, dst_t, zeros)

    y, s = _first_call(features, Wneigh[0], Wself[0], bb[0])
    a0, a1 = run_sc(y)
    residual = None
    for i in range(1, 13):
        y, s, h = _mid_call(s, a0, a1, Wneigh[i], Wself[i], bb[i])
        if i == 1:
            residual = h
        a0, a1 = run_sc(y)
    y, s, aux = _pre_call(s, a0, a1, residual, Wneigh[13], Wself[13], bb[13])
    a0, a1 = run_sc(y)
    vertices = _fin_call(s, a0, a1)
    return (vertices, aux)


# speedup vs baseline: 1.1955x; 1.0084x over previous
import functools

import jax
import jax.numpy as jnp
from jax import lax
from jax.experimental import pallas as pl
from jax.experimental.pallas import tpu as pltpu
from jax.experimental.pallas import tpu_sc as plsc

NN = 10000
EE = 320000
DD = 128
HH = 64
NCORE = 2
NTILE = 16
NPAD = 10240
RPT = NPAD // NTILE
EPT = EE // NTILE
KCH = 128
NCHUNK = -(-EPT // KCH)
EPAD = NCHUNK * KCH
BN = 2000
GRID = NN // BN


NBUF = 3


def _sc_agg_body(y_hbm, src_hbm, dst_hbm, zeros_hbm, out0_hbm, out1_hbm,
                 sidx, didx, rows0, rows1, rows2, agg_sp, sem0, sem1, sem2):
    c = lax.axis_index("c")
    s = lax.axis_index("s")
    rows = (rows0, rows1, rows2)
    sems = (sem0, sem1, sem2)

    pltpu.sync_copy(src_hbm.at[s], sidx)
    pltpu.sync_copy(dst_hbm.at[s], didx)
    pltpu.sync_copy(zeros_hbm, agg_sp.at[pl.ds(s * RPT, RPT)])
    plsc.subcore_barrier()

    def start(i, b):
        pltpu.async_copy(y_hbm.at[c].at[sidx.at[i]], rows[b], sems[b])

    def wait(b):
        pltpu.make_async_copy(y_hbm.at[0].at[pl.ds(0, KCH)], rows[b],
                              sems[b]).wait()

    def process(g, b):
        wait(b)
        pltpu.sync_copy(rows[b], agg_sp.at[didx.at[g]], add=True)

        @pl.when(g + NBUF < NCHUNK)
        def _():
            start(g + NBUF, b)

    for b in range(NBUF):
        start(b, b)

    def chunk_trip(t, carry):
        g = t * NBUF
        for b in range(NBUF):
            process(g + b, b)
        return carry

    lax.fori_loop(0, NCHUNK // NBUF, chunk_trip, 0)
    for r in range(NCHUNK - NCHUNK % NBUF, NCHUNK):
        process(r, r % NBUF)

    plsc.subcore_barrier()

    @pl.when(c == 0)
    def _():
        pltpu.sync_copy(agg_sp.at[pl.ds(s * RPT, RPT)],
                        out0_hbm.at[pl.ds(s * RPT, RPT)])

    @pl.when(c == 1)
    def _():
        pltpu.sync_copy(agg_sp.at[pl.ds(s * RPT, RPT)],
                        out1_hbm.at[pl.ds(s * RPT, RPT)])


_sc_agg = functools.partial(
    pl.kernel,
    mesh=plsc.VectorSubcoreMesh(core_axis_name="c", subcore_axis_name="s"),
    out_type=[jax.ShapeDtypeStruct((NPAD, HH), jnp.float32),
              jax.ShapeDtypeStruct((NPAD, HH), jnp.float32)],
    compiler_params=pltpu.CompilerParams(use_tc_tiling_on_sc=False),
    scratch_types=[
        pltpu.VMEM((NCHUNK, KCH), jnp.int32),
        pltpu.VMEM((NCHUNK, KCH), jnp.int32),
        pltpu.VMEM((KCH, HH), jnp.float32),
        pltpu.VMEM((KCH, HH), jnp.float32),
        pltpu.VMEM((KCH, HH), jnp.float32),
        pltpu.VMEM_SHARED((NPAD, HH), jnp.float32),
        pltpu.SemaphoreType.DMA,
        pltpu.SemaphoreType.DMA,
        pltpu.SemaphoreType.DMA,
    ],
)(_sc_agg_body)


def _pack_y(y, y_ref):
    y_ref[0] = y[:, :HH]
    y_ref[1] = y[:, HH:]


def _unpack_agg(a0_ref, a1_ref):
    return jnp.concatenate([a0_ref[...], a1_ref[...]], axis=1)


def _tc_first(x_ref, wn_ref, ws_ref, b_ref, y_ref, s_ref):
    x = x_ref[...]
    _pack_y(jnp.dot(x, wn_ref[...], preferred_element_type=jnp.float32), y_ref)
    s_ref[...] = jnp.dot(x, ws_ref[...], preferred_element_type=jnp.float32) + b_ref[...]


def _tc_mid(sp_ref, a0_ref, a1_ref, wn_ref, ws_ref, b_ref, y_ref, s_ref, h_ref):
    h = jnp.maximum(sp_ref[...] + _unpack_agg(a0_ref, a1_ref), 0.0)
    h_ref[...] = h
    _pack_y(jnp.dot(h, wn_ref[...], preferred_element_type=jnp.float32), y_ref)
    s_ref[...] = jnp.dot(h, ws_ref[...], preferred_element_type=jnp.float32) + b_ref[...]


def _tc_pre(sp_ref, a0_ref, a1_ref, res_ref, wn_ref, ws_ref, b_ref,
            y_ref, s_ref, aux_ref):
    h = jnp.maximum(sp_ref[...] + _unpack_agg(a0_ref, a1_ref), 0.0)
    aux_ref[...] = h
    x = res_ref[...] + h
    _pack_y(jnp.dot(x, wn_ref[...], preferred_element_type=jnp.float32), y_ref)
    s_ref[...] = jnp.dot(x, ws_ref[...], preferred_element_type=jnp.float32) + b_ref[...]


def _tc_fin(sp_ref, a0_ref, a1_ref, o_ref):
    o_ref[...] = sp_ref[...] + _unpack_agg(a0_ref, a1_ref)


_row = lambda r: (r, 0)
_full = lambda r: (0, 0)
_ND = pl.BlockSpec((BN, DD), _row)
_APK = pl.BlockSpec((BN, HH), _row)
_WB = pl.BlockSpec((DD, DD), _full)
_BB = pl.BlockSpec((1, DD), _full)
_Y = pl.BlockSpec((2, BN, HH), lambda r: (0, r, 0))

_sd_ND = jax.ShapeDtypeStruct((NN, DD), jnp.float32)
_sd_Y = jax.ShapeDtypeStruct((2, NN, HH), jnp.float32)

_first_call = pl.pallas_call(
    _tc_first, grid=(GRID,),
    in_specs=[_ND, _WB, _WB, _BB],
    out_specs=[_Y, _ND],
    out_shape=[_sd_Y, _sd_ND],
)

_mid_call = pl.pallas_call(
    _tc_mid, grid=(GRID,),
    in_specs=[_ND, _APK, _APK, _WB, _WB, _BB],
    out_specs=[_Y, _ND, _ND],
    out_shape=[_sd_Y, _sd_ND, _sd_ND],
)

_pre_call = pl.pallas_call(
    _tc_pre, grid=(GRID,),
    in_specs=[_ND, _APK, _APK, _ND, _WB, _WB, _BB],
    out_specs=[_Y, _ND, _ND],
    out_shape=[_sd_Y, _sd_ND, _sd_ND],
)

_fin_call = pl.pallas_call(
    _tc_fin, grid=(GRID,),
    in_specs=[_ND, _APK, _APK],
    out_specs=_ND,
    out_shape=_sd_ND,
)


def kernel(features, edges, Wself, Wneigh, b):
    src = edges[0].astype(jnp.int32)
    dst = edges[1].astype(jnp.int32)
    pad = EPAD - EPT
    src_t = jnp.pad(src.reshape(NTILE, EPT), ((0, 0), (0, pad)))
    src_t = src_t.reshape(NTILE, NCHUNK, KCH)
    dst_t = jnp.pad(dst.reshape(NTILE, EPT), ((0, 0), (0, pad)),
                    constant_values=NN)
    dst_t = dst_t.reshape(NTILE, NCHUNK, KCH)
    zeros = jnp.zeros((RPT, HH), jnp.float32)
    bb = b.reshape(-1, 1, DD)

    def run_sc(y):
        return _sc_agg(y, src_t, dst_t, zeros)

    y, s = _first_call(features, Wneigh[0], Wself[0], bb[0])
    a0, a1 = run_sc(y)
    residual = None
    for i in range(1, 13):
        y, s, h = _mid_call(s, a0, a1, Wneigh[i], Wself[i], bb[i])
        if i == 1:
            residual = h
        a0, a1 = run_sc(y)
    y, s, aux = _pre_call(s, a0, a1, residual, Wneigh[13], Wself[13], bb[13])
    a0, a1 = run_sc(y)
    vertices = _fin_call(s, a0, a1)
    return (vertices, aux)


# speedup vs baseline: 1.2166x; 1.0177x over previous
import functools

import jax
import jax.numpy as jnp
from jax import lax
from jax.experimental import pallas as pl
from jax.experimental.pallas import tpu as pltpu
from jax.experimental.pallas import tpu_sc as plsc

NN = 10000
EE = 320000
DD = 128
HH = 64
NCORE = 2
NTILE = 16
NPAD = 10240
RPT = NPAD // NTILE
EPT = EE // NTILE
KCH = 128
NCHUNK = -(-EPT // KCH)
EPAD = NCHUNK * KCH
BN = 5000
GRID = NN // BN


NBUF = 3


def _sc_agg_body(y_hbm, src_hbm, dst_hbm, zeros_hbm, out0_hbm, out1_hbm,
                 sidx, didx, rows0, rows1, rows2, agg_sp, sem0, sem1, sem2):
    c = lax.axis_index("c")
    s = lax.axis_index("s")
    rows = (rows0, rows1, rows2)
    sems = (sem0, sem1, sem2)

    pltpu.sync_copy(src_hbm.at[s], sidx)
    pltpu.sync_copy(dst_hbm.at[s], didx)
    pltpu.sync_copy(zeros_hbm, agg_sp.at[pl.ds(s * RPT, RPT)])
    plsc.subcore_barrier()

    def start(i, b):
        pltpu.async_copy(y_hbm.at[c].at[sidx.at[i]], rows[b], sems[b])

    def wait(b):
        pltpu.make_async_copy(y_hbm.at[0].at[pl.ds(0, KCH)], rows[b],
                              sems[b]).wait()

    def process(g, b):
        wait(b)
        pltpu.sync_copy(rows[b], agg_sp.at[didx.at[g]], add=True)

        @pl.when(g + NBUF < NCHUNK)
        def _():
            start(g + NBUF, b)

    for b in range(NBUF):
        start(b, b)

    def chunk_trip(t, carry):
        g = t * NBUF
        for b in range(NBUF):
            process(g + b, b)
        return carry

    lax.fori_loop(0, NCHUNK // NBUF, chunk_trip, 0)
    for r in range(NCHUNK - NCHUNK % NBUF, NCHUNK):
        process(r, r % NBUF)

    plsc.subcore_barrier()

    @pl.when(c == 0)
    def _():
        pltpu.sync_copy(agg_sp.at[pl.ds(s * RPT, RPT)],
                        out0_hbm.at[pl.ds(s * RPT, RPT)])

    @pl.when(c == 1)
    def _():
        pltpu.sync_copy(agg_sp.at[pl.ds(s * RPT, RPT)],
                        out1_hbm.at[pl.ds(s * RPT, RPT)])


_sc_agg = functools.partial(
    pl.kernel,
    mesh=plsc.VectorSubcoreMesh(core_axis_name="c", subcore_axis_name="s"),
    out_type=[jax.ShapeDtypeStruct((NPAD, HH), jnp.float32),
              jax.ShapeDtypeStruct((NPAD, HH), jnp.float32)],
    compiler_params=pltpu.CompilerParams(use_tc_tiling_on_sc=False),
    scratch_types=[
        pltpu.VMEM((NCHUNK, KCH), jnp.int32),
        pltpu.VMEM((NCHUNK, KCH), jnp.int32),
        pltpu.VMEM((KCH, HH), jnp.float32),
        pltpu.VMEM((KCH, HH), jnp.float32),
        pltpu.VMEM((KCH, HH), jnp.float32),
        pltpu.VMEM_SHARED((NPAD, HH), jnp.float32),
        pltpu.SemaphoreType.DMA,
        pltpu.SemaphoreType.DMA,
        pltpu.SemaphoreType.DMA,
    ],
)(_sc_agg_body)


def _pack_y(y, y_ref):
    y_ref[0] = y[:, :HH]
    y_ref[1] = y[:, HH:]


def _unpack_agg(a0_ref, a1_ref):
    return jnp.concatenate([a0_ref[...], a1_ref[...]], axis=1)


def _tc_first(x_ref, wn_ref, ws_ref, b_ref, y_ref, s_ref):
    x = x_ref[...]
    _pack_y(jnp.dot(x, wn_ref[...], preferred_element_type=jnp.float32), y_ref)
    s_ref[...] = jnp.dot(x, ws_ref[...], preferred_element_type=jnp.float32) + b_ref[...]


def _tc_mid(sp_ref, a0_ref, a1_ref, wn_ref, ws_ref, b_ref, y_ref, s_ref, h_ref):
    h = jnp.maximum(sp_ref[...] + _unpack_agg(a0_ref, a1_ref), 0.0)
    h_ref[...] = h
    _pack_y(jnp.dot(h, wn_ref[...], preferred_element_type=jnp.float32), y_ref)
    s_ref[...] = jnp.dot(h, ws_ref[...], preferred_element_type=jnp.float32) + b_ref[...]


def _tc_pre(sp_ref, a0_ref, a1_ref, res_ref, wn_ref, ws_ref, b_ref,
            y_ref, s_ref, aux_ref):
    h = jnp.maximum(sp_ref[...] + _unpack_agg(a0_ref, a1_ref), 0.0)
    aux_ref[...] = h
    x = res_ref[...] + h
    _pack_y(jnp.dot(x, wn_ref[...], preferred_element_type=jnp.float32), y_ref)
    s_ref[...] = jnp.dot(x, ws_ref[...], preferred_element_type=jnp.float32) + b_ref[...]


def _tc_fin(sp_ref, a0_ref, a1_ref, o_ref):
    o_ref[...] = sp_ref[...] + _unpack_agg(a0_ref, a1_ref)


_row = lambda r: (r, 0)
_full = lambda r: (0, 0)
_ND = pl.BlockSpec((BN, DD), _row)
_APK = pl.BlockSpec((BN, HH), _row)
_WB = pl.BlockSpec((DD, DD), _full)
_BB = pl.BlockSpec((1, DD), _full)
_Y = pl.BlockSpec((2, BN, HH), lambda r: (0, r, 0))

_sd_ND = jax.ShapeDtypeStruct((NN, DD), jnp.float32)
_sd_Y = jax.ShapeDtypeStruct((2, NN, HH), jnp.float32)

_first_call = pl.pallas_call(
    _tc_first, grid=(GRID,),
    in_specs=[_ND, _WB, _WB, _BB],
    out_specs=[_Y, _ND],
    out_shape=[_sd_Y, _sd_ND],
)

_mid_call = pl.pallas_call(
    _tc_mid, grid=(GRID,),
    in_specs=[_ND, _APK, _APK, _WB, _WB, _BB],
    out_specs=[_Y, _ND, _ND],
    out_shape=[_sd_Y, _sd_ND, _sd_ND],
)

_pre_call = pl.pallas_call(
    _tc_pre, grid=(GRID,),
    in_specs=[_ND, _APK, _APK, _ND, _WB, _WB, _BB],
    out_specs=[_Y, _ND, _ND],
    out_shape=[_sd_Y, _sd_ND, _sd_ND],
)

_fin_call = pl.pallas_call(
    _tc_fin, grid=(GRID,),
    in_specs=[_ND, _APK, _APK],
    out_specs=_ND,
    out_shape=_sd_ND,
)


def kernel(features, edges, Wself, Wneigh, b):
    src = edges[0].astype(jnp.int32)
    dst = edges[1].astype(jnp.int32)
    pad = EPAD - EPT
    src_t = jnp.pad(src.reshape(NTILE, EPT), ((0, 0), (0, pad)))
    src_t = src_t.reshape(NTILE, NCHUNK, KCH)
    dst_t = jnp.pad(dst.reshape(NTILE, EPT), ((0, 0), (0, pad)),
                    constant_values=NN)
    dst_t = dst_t.reshape(NTILE, NCHUNK, KCH)
    zeros = jnp.zeros((RPT, HH), jnp.float32)
    bb = b.reshape(-1, 1, DD)

    def run_sc(y):
        return _sc_agg(y, src_t, dst_t, zeros)

    y, s = _first_call(features, Wneigh[0], Wself[0], bb[0])
    a0, a1 = run_sc(y)
    residual = None
    for i in range(1, 13):
        y, s, h = _mid_call(s, a0, a1, Wneigh[i], Wself[i], bb[i])
        if i == 1:
            residual = h
        a0, a1 = run_sc(y)
    y, s, aux = _pre_call(s, a0, a1, residual, Wneigh[13], Wself[13], bb[13])
    a0, a1 = run_sc(y)
    vertices = _fin_call(s, a0, a1)
    return (vertices, aux)
